# Initial kernel scaffold; baseline (speedup 1.0000x reference)
import functools

import jax
import jax.numpy as jnp
from jax import lax
from jax.experimental import pallas as pl
from jax.experimental.pallas import tpu as pltpu
from jax.experimental.pallas import tpu_sc as plsc

N_PAD = 10240
D_IN = 128
HC = 128
H = 4
K = 128
NCHUNK = 81
NW = 32
ROWS_PT = N_PAD // 16


def _proj_body(x_ref, w_ref, wa_ref, h_ref, a_ref):
    xv = x_ref[...]
    xe = jnp.where(xv > 0, xv, jnp.expm1(xv))
    h_ref[...] = jnp.dot(xe, w_ref[...], preferred_element_type=jnp.float32)
    a_ref[...] = jnp.dot(xe, wa_ref[...], preferred_element_type=jnp.float32)


def _project(x_pad, W, Wa):
    blk = 512
    return pl.pallas_call(
        _proj_body,
        grid=(N_PAD // blk,),
        in_specs=[
            pl.BlockSpec((blk, D_IN), lambda i: (i, 0)),
            pl.BlockSpec((D_IN, HC), lambda i: (0, 0)),
            pl.BlockSpec((D_IN, 8), lambda i: (0, 0)),
        ],
        out_specs=[
            pl.BlockSpec((blk, HC), lambda i: (i, 0)),
            pl.BlockSpec((blk, 8), lambda i: (i, 0)),
        ],
        out_shape=[
            jax.ShapeDtypeStruct((N_PAD, HC), jnp.float32),
            jax.ShapeDtypeStruct((N_PAD, 8), jnp.float32),
        ],
    )(x_pad, W, Wa)


def _agg_body(h_hbm, a_hbm, src_hbm, dst_hbm, num_hbm, den_hbm,
              a_vmem, src_vmem, dst_vmem, rows_buf, w_rows, acc, den_acc, sem):
    cid = lax.axis_index("c")
    sid = lax.axis_index("s")
    wid = sid * 2 + cid
    zeros16 = jnp.zeros((16,), jnp.float32)
    iota16 = lax.iota(jnp.int32, 16)

    pltpu.sync_copy(a_hbm, a_vmem)
    pltpu.sync_copy(src_hbm.at[wid], src_vmem)
    pltpu.sync_copy(dst_hbm.at[wid], dst_vmem)

    def _zero_row(r, _):
        for q in range(8):
            rows_buf[r, pl.ds(q * 16, 16)] = zeros16
        w_rows[r, pl.ds(0, 16)] = zeros16
        return 0
    lax.fori_loop(0, K, _zero_row, 0)
    for j in range(ROWS_PT // K):
        base = sid * ROWS_PT + j * K
        pltpu.sync_copy(rows_buf, acc.at[pl.ds(base, K)])
        pltpu.sync_copy(w_rows, den_acc.at[pl.ds(base, K)])
    plsc.subcore_barrier()

    def _chunk(c, _):
        pltpu.async_copy(h_hbm.at[src_vmem.at[c]], rows_buf, sem).wait()
        for g in range(K // 16):
            sv = src_vmem[c, pl.ds(g * 16, 16)]
            dv = dst_vmem[c, pl.ds(g * 16, 16)]
            for head in range(H):
                asv = plsc.load_gather(a_vmem, [sv, jnp.full((16,), head, jnp.int32)])
                adv = plsc.load_gather(a_vmem, [dv, jnp.full((16,), head + 4, jnp.int32)])
                al = asv + adv
                al = jnp.maximum(al, al * 0.2)
                plsc.store_scatter(
                    w_rows,
                    [g * 16 + iota16, jnp.full((16,), head, jnp.int32)],
                    jnp.exp(al))
        def _scale(e, _):
            for head in range(H):
                ws = w_rows[e, head]
                for q in range(2):
                    col = head * 32 + q * 16
                    rows_buf[e, pl.ds(col, 16)] = rows_buf[e, pl.ds(col, 16)] * ws
            return 0
        lax.fori_loop(0, K, _scale, 0)
        pltpu.sync_copy(rows_buf, acc.at[dst_vmem.at[c]], add=True)
        pltpu.sync_copy(w_rows, den_acc.at[dst_vmem.at[c]], add=True)
        return 0
    lax.fori_loop(0, NCHUNK, _chunk, 0)
    plsc.subcore_barrier()

    base = sid * ROWS_PT
    pltpu.sync_copy(acc.at[pl.ds(base, ROWS_PT)], num_hbm.at[cid, pl.ds(base, ROWS_PT)])
    pltpu.sync_copy(den_acc.at[pl.ds(base, ROWS_PT)], den_hbm.at[cid, pl.ds(base, ROWS_PT)])


def _aggregate(h, a, src_p, dst_p):
    mesh = plsc.VectorSubcoreMesh(core_axis_name="c", subcore_axis_name="s",
                                  num_cores=2, num_subcores=16)
    return pl.kernel(
        _agg_body,
        out_type=[
            jax.ShapeDtypeStruct((2, N_PAD, HC), jnp.float32),
            jax.ShapeDtypeStruct((2, N_PAD, 16), jnp.float32),
        ],
        mesh=mesh,
        scratch_types=[
            pltpu.VMEM((N_PAD, 8), jnp.float32),
            pltpu.VMEM((NCHUNK, K), jnp.int32),
            pltpu.VMEM((NCHUNK, K), jnp.int32),
            pltpu.VMEM((K, HC), jnp.float32),
            pltpu.VMEM((K, 16), jnp.float32),
            pltpu.VMEM_SHARED((N_PAD, HC), jnp.float32),
            pltpu.VMEM_SHARED((N_PAD, 16), jnp.float32),
            pltpu.SemaphoreType.DMA,
        ],
    )(h, a, src_p, dst_p)


def _fin_body(n_ref, d_ref, rep_ref, b_ref, o_ref):
    num = n_ref[0] + n_ref[1]
    den = d_ref[0] + d_ref[1]
    denf = jnp.dot(den, rep_ref[...], preferred_element_type=jnp.float32)
    o_ref[...] = num / denf + b_ref[...]


def _finalize(num, den, Rep, bias2d):
    blk = 512
    return pl.pallas_call(
        _fin_body,
        grid=(N_PAD // blk,),
        in_specs=[
            pl.BlockSpec((2, blk, HC), lambda i: (0, i, 0)),
            pl.BlockSpec((2, blk, 16), lambda i: (0, i, 0)),
            pl.BlockSpec((16, HC), lambda i: (0, 0)),
            pl.BlockSpec((1, HC), lambda i: (0, 0)),
        ],
        out_specs=pl.BlockSpec((blk, HC), lambda i: (i, 0)),
        out_shape=jax.ShapeDtypeStruct((N_PAD, HC), jnp.float32),
    )(num, den, Rep, bias2d)


@jax.jit
def kernel(x, edge_index, W, att_src, att_dst, bias):
    n = x.shape[0]
    x_pad = jnp.zeros((N_PAD, D_IN), jnp.float32).at[:n].set(x)
    W3 = W.reshape(D_IN, H, HC // H)
    Ws = jnp.einsum("dhc,hc->dh", W3, att_src)
    Wd = jnp.einsum("dhc,hc->dh", W3, att_dst)
    Wa = jnp.concatenate([Ws, Wd], axis=1)

    h, a = _project(x_pad, W, Wa)

    loops = jnp.arange(n, dtype=jnp.int32)
    src = jnp.concatenate([edge_index[0].astype(jnp.int32), loops])
    dst = jnp.concatenate([edge_index[1].astype(jnp.int32), loops])
    e_tot = src.shape[0]
    ep = NW * NCHUNK * K
    fill = jnp.full((ep - e_tot,), N_PAD - 1, jnp.int32)
    src_p = jnp.concatenate([src, fill]).reshape(NW, NCHUNK, K)
    dst_p = jnp.concatenate([dst, fill]).reshape(NW, NCHUNK, K)

    num, den = _aggregate(h, a, src_p, dst_p)

    rep = jnp.zeros((16, HC), jnp.float32)
    rep = rep.at[jnp.repeat(jnp.arange(H), HC // H),
                 jnp.arange(HC)].set(1.0)
    out = _finalize(num, den, rep, bias.reshape(1, HC))
    return out[:n]


# speedup vs baseline: 61.4708x; 61.4708x over previous
import jax
import jax.numpy as jnp
from jax import lax
from jax.experimental import pallas as pl
from jax.experimental.pallas import tpu as pltpu
from jax.experimental.pallas import tpu_sc as plsc

N_PAD = 10016
D_IN = 128
HC = 128
HHC = HC // 2
H = 4
K = 48
NCHUNK = 432
NT = 16
ROWS_PT = N_PAD // NT
EP = NT * NCHUNK * K


def _proj_body(x_ref, w_ref, wa_ref, hl_ref, hr_ref, a_ref):
    xv = x_ref[...]
    xe = jnp.where(xv > 0, xv, jnp.exp(xv) - 1.0)
    w = w_ref[...]
    hl_ref[...] = jnp.dot(xe, w[:, :HHC], preferred_element_type=jnp.float32)
    hr_ref[...] = jnp.dot(xe, w[:, HHC:], preferred_element_type=jnp.float32)
    a_ref[...] = jnp.dot(xe, wa_ref[...], preferred_element_type=jnp.float32)


def _project(x_pad, W, Wa):
    blk = 2504
    return pl.pallas_call(
        _proj_body,
        grid=(N_PAD // blk,),
        in_specs=[
            pl.BlockSpec((blk, D_IN), lambda i: (i, 0)),
            pl.BlockSpec((D_IN, HC), lambda i: (0, 0)),
            pl.BlockSpec((D_IN, 8), lambda i: (0, 0)),
        ],
        out_specs=[
            pl.BlockSpec((blk, HHC), lambda i: (i, 0)),
            pl.BlockSpec((blk, HHC), lambda i: (i, 0)),
            pl.BlockSpec((blk, 8), lambda i: (i, 0)),
        ],
        out_shape=[
            jax.ShapeDtypeStruct((N_PAD, HHC), jnp.float32),
            jax.ShapeDtypeStruct((N_PAD, HHC), jnp.float32),
            jax.ShapeDtypeStruct((N_PAD, 8), jnp.float32),
        ],
    )(x_pad, W, Wa)


def _agg_body(hl_hbm, hr_hbm, a_hbm, src_hbm, dst_hbm, num_hbm, den_hbm,
              a_vmem, src_vmem, dst_vmem, rows_buf, w_pl, w_rows, acc, den_acc, sem):
    cid = lax.axis_index("c")
    sid = lax.axis_index("s")
    zeros16 = jnp.zeros((16,), jnp.float32)
    iota16 = lax.iota(jnp.int32, 16)

    pltpu.sync_copy(a_hbm.at[cid], a_vmem)
    pltpu.sync_copy(src_hbm.at[sid], src_vmem)
    pltpu.sync_copy(dst_hbm.at[sid], dst_vmem)

    def _zero_row(r, _):
        for q in range(HHC // 16):
            rows_buf[r, pl.ds(q * 16, 16)] = zeros16
        return 0
    lax.fori_loop(0, K, _zero_row, 0)
    for g in range(K // 16):
        for cc in range(8):
            plsc.store_scatter(w_rows,
                               [g * 16 + iota16, jnp.full((16,), cc, jnp.int32)],
                               zeros16)
    for j in range(ROWS_PT // K):
        pltpu.sync_copy(rows_buf, acc.at[pl.ds(sid * ROWS_PT + j * K, K)])
        pltpu.sync_copy(w_rows, den_acc.at[pl.ds(sid * ROWS_PT + j * K, K)])
    _rem = ROWS_PT % K
    if _rem:
        pltpu.sync_copy(rows_buf.at[pl.ds(0, _rem)],
                        acc.at[pl.ds(sid * ROWS_PT + (ROWS_PT // K) * K, _rem)])
        pltpu.sync_copy(w_rows.at[pl.ds(0, _rem)],
                        den_acc.at[pl.ds(sid * ROWS_PT + (ROWS_PT // K) * K, _rem)])
    plsc.subcore_barrier()

    def _chunk(c, _):
        @pl.when(cid == 0)
        def _():
            pltpu.async_copy(hl_hbm.at[src_vmem.at[c]], rows_buf, sem).wait()

        @pl.when(cid == 1)
        def _():
            pltpu.async_copy(hr_hbm.at[src_vmem.at[c]], rows_buf, sem).wait()
        for g in range(K // 16):
            sv = src_vmem[c, pl.ds(g * 16, 16)]
            dv = dst_vmem[c, pl.ds(g * 16, 16)]
            for hh in range(2):
                asv = plsc.load_gather(a_vmem, [sv + hh * N_PAD])
                adv = plsc.load_gather(a_vmem, [dv + (2 + hh) * N_PAD])
                al = asv + adv
                al = jnp.maximum(al, al * 0.2)
                w = jnp.exp(al)
                w_pl[hh, pl.ds(g * 16, 16)] = w
                plsc.store_scatter(
                    w_rows,
                    [g * 16 + iota16, jnp.full((16,), hh, jnp.int32)],
                    w)
        for g in range(K // 16):
            w0 = w_pl[0, pl.ds(g * 16, 16)]
            w1 = w_pl[1, pl.ds(g * 16, 16)]
            for j in range(16):
                e = g * 16 + j
                for q in range(4):
                    ws = w0[j] if q < 2 else w1[j]
                    rows_buf[e, pl.ds(q * 16, 16)] = rows_buf[e, pl.ds(q * 16, 16)] * ws
        pltpu.sync_copy(rows_buf, acc.at[dst_vmem.at[c]], add=True)
        pltpu.sync_copy(w_rows, den_acc.at[dst_vmem.at[c]], add=True)
        return 0
    lax.fori_loop(0, NCHUNK, _chunk, 0)
    plsc.subcore_barrier()

    base = sid * ROWS_PT
    pltpu.sync_copy(acc.at[pl.ds(base, ROWS_PT)], num_hbm.at[cid, pl.ds(base, ROWS_PT)])
    pltpu.sync_copy(den_acc.at[pl.ds(base, ROWS_PT)], den_hbm.at[cid, pl.ds(base, ROWS_PT)])


def _aggregate(hl, hr, a_planes, src_p, dst_p):
    mesh = plsc.VectorSubcoreMesh(core_axis_name="c", subcore_axis_name="s",
                                  num_cores=2, num_subcores=NT)
    return pl.kernel(
        _agg_body,
        out_type=[
            jax.ShapeDtypeStruct((2, N_PAD, HHC), jnp.float32),
            jax.ShapeDtypeStruct((2, N_PAD, 8), jnp.float32),
        ],
        mesh=mesh,
        compiler_params=pltpu.CompilerParams(needs_layout_passes=False, use_tc_tiling_on_sc=False),
        scratch_types=[
            pltpu.VMEM((4 * N_PAD,), jnp.float32),
            pltpu.VMEM((NCHUNK, K), jnp.int32),
            pltpu.VMEM((NCHUNK, K), jnp.int32),
            pltpu.VMEM((K, HHC), jnp.float32),
            pltpu.VMEM((2, K), jnp.float32),
            pltpu.VMEM((K, 8), jnp.float32),
            pltpu.VMEM_SHARED((N_PAD, HHC), jnp.float32),
            pltpu.VMEM_SHARED((N_PAD, 8), jnp.float32),
            pltpu.SemaphoreType.DMA,
        ],
    )(hl, hr, a_planes, src_p, dst_p)


def _fin_body(n_ref, d_ref, rep_ref, b_ref, o_ref):
    num = jnp.concatenate([n_ref[0], n_ref[1]], axis=1)
    den = jnp.concatenate([d_ref[0], d_ref[1]], axis=1)
    denf = jnp.dot(den, rep_ref[...], preferred_element_type=jnp.float32)
    o_ref[...] = num / denf + b_ref[...]


def _finalize(num, den, Rep, bias2d):
    blk = 2504
    return pl.pallas_call(
        _fin_body,
        grid=(N_PAD // blk,),
        in_specs=[
            pl.BlockSpec((2, blk, HHC), lambda i: (0, i, 0)),
            pl.BlockSpec((2, blk, 8), lambda i: (0, i, 0)),
            pl.BlockSpec((16, HC), lambda i: (0, 0)),
            pl.BlockSpec((1, HC), lambda i: (0, 0)),
        ],
        out_specs=pl.BlockSpec((blk, HC), lambda i: (i, 0)),
        out_shape=jax.ShapeDtypeStruct((N_PAD, HC), jnp.float32),
    )(num, den, Rep, bias2d)


@jax.jit
def kernel(x, edge_index, W, att_src, att_dst, bias):
    n = x.shape[0]
    x_pad = jnp.zeros((N_PAD, D_IN), jnp.float32).at[:n].set(x)
    W3 = W.reshape(D_IN, H, HC // H)
    Ws = jnp.einsum("dhc,hc->dh", W3, att_src)
    Wd = jnp.einsum("dhc,hc->dh", W3, att_dst)
    Wa = jnp.concatenate([Ws, Wd], axis=1)

    hl, hr, a = _project(x_pad, W, Wa)

    aT = a.T
    a_planes = jnp.stack([
        jnp.concatenate([aT[0], aT[1], aT[4], aT[5]]),
        jnp.concatenate([aT[2], aT[3], aT[6], aT[7]]),
    ])

    loops = jnp.arange(n, dtype=jnp.int32)
    src = jnp.concatenate([edge_index[0].astype(jnp.int32), loops])
    dst = jnp.concatenate([edge_index[1].astype(jnp.int32), loops])
    e_tot = src.shape[0]
    fill = jnp.full((EP - e_tot,), N_PAD - 1, jnp.int32)
    src_p = jnp.concatenate([src, fill]).reshape(NT, NCHUNK, K)
    dst_p = jnp.concatenate([dst, fill]).reshape(NT, NCHUNK, K)

    num, den = _aggregate(hl, hr, a_planes, src_p, dst_p)

    rep = jnp.zeros((16, HC), jnp.float32)
    for c in range(2):
        for hh in range(2):
            head = 2 * c + hh
            rep = rep.at[8 * c + hh, 32 * head:32 * (head + 1)].set(1.0)
    out = _finalize(num, den, rep, bias.reshape(1, HC))
    return out[:n]


# speedup vs baseline: 65.7712x; 1.0700x over previous
import jax
import jax.numpy as jnp
from jax import lax
from jax.experimental import pallas as pl
from jax.experimental.pallas import tpu as pltpu
from jax.experimental.pallas import tpu_sc as plsc

N_PAD = 10016
D_IN = 128
HC = 128
HHC = HC // 2
H = 4
K = 48
NCHUNK = 432
NT = 16
ROWS_PT = N_PAD // NT
EP = NT * NCHUNK * K


def _proj_body(x_ref, w_ref, wa_ref, hl_ref, hr_ref, a_ref):
    xv = x_ref[...]
    xe = jnp.where(xv > 0, xv, jnp.exp(xv) - 1.0)
    w = w_ref[...]
    hl_ref[...] = jnp.dot(xe, w[:, :HHC], preferred_element_type=jnp.float32)
    hr_ref[...] = jnp.dot(xe, w[:, HHC:], preferred_element_type=jnp.float32)
    a_ref[...] = jnp.dot(xe, wa_ref[...], preferred_element_type=jnp.float32)


def _project(x_pad, W, Wa):
    blk = 2504
    return pl.pallas_call(
        _proj_body,
        grid=(N_PAD // blk,),
        in_specs=[
            pl.BlockSpec((blk, D_IN), lambda i: (i, 0)),
            pl.BlockSpec((D_IN, HC), lambda i: (0, 0)),
            pl.BlockSpec((D_IN, 8), lambda i: (0, 0)),
        ],
        out_specs=[
            pl.BlockSpec((blk, HHC), lambda i: (i, 0)),
            pl.BlockSpec((blk, HHC), lambda i: (i, 0)),
            pl.BlockSpec((blk, 8), lambda i: (i, 0)),
        ],
        out_shape=[
            jax.ShapeDtypeStruct((N_PAD, HHC), jnp.float32),
            jax.ShapeDtypeStruct((N_PAD, HHC), jnp.float32),
            jax.ShapeDtypeStruct((N_PAD, 8), jnp.float32),
        ],
    )(x_pad, W, Wa)


def _agg_body(hl_hbm, hr_hbm, a_hbm, src_hbm, dst_hbm, num_hbm, den_hbm,
              a_vmem, src_vmem, dst_vmem, rows_buf, w_pl, w_rows, acc, den_acc, sem):
    cid = lax.axis_index("c")
    sid = lax.axis_index("s")
    zeros16 = jnp.zeros((16,), jnp.float32)
    iota16 = lax.iota(jnp.int32, 16)

    pltpu.sync_copy(a_hbm.at[cid], a_vmem)
    pltpu.sync_copy(src_hbm.at[sid], src_vmem)
    pltpu.sync_copy(dst_hbm.at[sid], dst_vmem)

    def _zero_row(r, _):
        for q in range(HHC // 16):
            rows_buf[r, pl.ds(q * 16, 16)] = zeros16
        return 0
    lax.fori_loop(0, K, _zero_row, 0)
    for g in range(K // 16):
        for cc in range(8):
            plsc.store_scatter(w_rows,
                               [g * 16 + iota16, jnp.full((16,), cc, jnp.int32)],
                               zeros16)
    for j in range(ROWS_PT // K):
        pltpu.sync_copy(rows_buf, acc.at[pl.ds(sid * ROWS_PT + j * K, K)])
        pltpu.sync_copy(w_rows, den_acc.at[pl.ds(sid * ROWS_PT + j * K, K)])
    _rem = ROWS_PT % K
    if _rem:
        pltpu.sync_copy(rows_buf.at[pl.ds(0, _rem)],
                        acc.at[pl.ds(sid * ROWS_PT + (ROWS_PT // K) * K, _rem)])
        pltpu.sync_copy(w_rows.at[pl.ds(0, _rem)],
                        den_acc.at[pl.ds(sid * ROWS_PT + (ROWS_PT // K) * K, _rem)])
    plsc.subcore_barrier()

    def _chunk(c, _):
        @pl.when(cid == 0)
        def _():
            pltpu.async_copy(hl_hbm.at[src_vmem.at[c]], rows_buf, sem)

        @pl.when(cid == 1)
        def _():
            pltpu.async_copy(hr_hbm.at[src_vmem.at[c]], rows_buf, sem)
        for g in range(K // 16):
            sv = src_vmem[c, pl.ds(g * 16, 16)]
            dv = dst_vmem[c, pl.ds(g * 16, 16)]
            for hh in range(2):
                asv = plsc.load_gather(a_vmem, [sv + hh * N_PAD])
                adv = plsc.load_gather(a_vmem, [dv + (2 + hh) * N_PAD])
                al = asv + adv
                al = jnp.maximum(al, al * 0.2)
                w = jnp.exp(al)
                w_pl[hh, pl.ds(g * 16, 16)] = w
                plsc.store_scatter(
                    w_rows,
                    [g * 16 + iota16, jnp.full((16,), hh, jnp.int32)],
                    w)
        pltpu.make_async_copy(hl_hbm.at[src_vmem.at[c]], rows_buf, sem).wait()
        for g in range(K // 16):
            w0 = w_pl[0, pl.ds(g * 16, 16)]
            w1 = w_pl[1, pl.ds(g * 16, 16)]
            for j in range(16):
                e = g * 16 + j
                for q in range(4):
                    ws = w0[j] if q < 2 else w1[j]
                    rows_buf[e, pl.ds(q * 16, 16)] = rows_buf[e, pl.ds(q * 16, 16)] * ws
        pltpu.sync_copy(rows_buf, acc.at[dst_vmem.at[c]], add=True)
        pltpu.sync_copy(w_rows, den_acc.at[dst_vmem.at[c]], add=True)
        return 0
    lax.fori_loop(0, NCHUNK, _chunk, 0)
    plsc.subcore_barrier()

    base = sid * ROWS_PT
    pltpu.sync_copy(acc.at[pl.ds(base, ROWS_PT)], num_hbm.at[cid, pl.ds(base, ROWS_PT)])
    pltpu.sync_copy(den_acc.at[pl.ds(base, ROWS_PT)], den_hbm.at[cid, pl.ds(base, ROWS_PT)])


def _aggregate(hl, hr, a_planes, src_p, dst_p):
    mesh = plsc.VectorSubcoreMesh(core_axis_name="c", subcore_axis_name="s",
                                  num_cores=2, num_subcores=NT)
    return pl.kernel(
        _agg_body,
        out_type=[
            jax.ShapeDtypeStruct((2, N_PAD, HHC), jnp.float32),
            jax.ShapeDtypeStruct((2, N_PAD, 8), jnp.float32),
        ],
        mesh=mesh,
        compiler_params=pltpu.CompilerParams(needs_layout_passes=False, use_tc_tiling_on_sc=False),
        scratch_types=[
            pltpu.VMEM((4 * N_PAD,), jnp.float32),
            pltpu.VMEM((NCHUNK, K), jnp.int32),
            pltpu.VMEM((NCHUNK, K), jnp.int32),
            pltpu.VMEM((K, HHC), jnp.float32),
            pltpu.VMEM((2, K), jnp.float32),
            pltpu.VMEM((K, 8), jnp.float32),
            pltpu.VMEM_SHARED((N_PAD, HHC), jnp.float32),
            pltpu.VMEM_SHARED((N_PAD, 8), jnp.float32),
            pltpu.SemaphoreType.DMA,
        ],
    )(hl, hr, a_planes, src_p, dst_p)


def _fin_body(n_ref, d_ref, rep_ref, b_ref, o_ref):
    num = jnp.concatenate([n_ref[0], n_ref[1]], axis=1)
    den = jnp.concatenate([d_ref[0], d_ref[1]], axis=1)
    denf = jnp.dot(den, rep_ref[...], preferred_element_type=jnp.float32)
    o_ref[...] = num / denf + b_ref[...]


def _finalize(num, den, Rep, bias2d):
    blk = 2504
    return pl.pallas_call(
        _fin_body,
        grid=(N_PAD // blk,),
        in_specs=[
            pl.BlockSpec((2, blk, HHC), lambda i: (0, i, 0)),
            pl.BlockSpec((2, blk, 8), lambda i: (0, i, 0)),
            pl.BlockSpec((16, HC), lambda i: (0, 0)),
            pl.BlockSpec((1, HC), lambda i: (0, 0)),
        ],
        out_specs=pl.BlockSpec((blk, HC), lambda i: (i, 0)),
        out_shape=jax.ShapeDtypeStruct((N_PAD, HC), jnp.float32),
    )(num, den, Rep, bias2d)


@jax.jit
def kernel(x, edge_index, W, att_src, att_dst, bias):
    n = x.shape[0]
    x_pad = jnp.zeros((N_PAD, D_IN), jnp.float32).at[:n].set(x)
    W3 = W.reshape(D_IN, H, HC // H)
    Ws = jnp.einsum("dhc,hc->dh", W3, att_src)
    Wd = jnp.einsum("dhc,hc->dh", W3, att_dst)
    Wa = jnp.concatenate([Ws, Wd], axis=1)

    hl, hr, a = _project(x_pad, W, Wa)

    aT = a.T
    a_planes = jnp.stack([
        jnp.concatenate([aT[0], aT[1], aT[4], aT[5]]),
        jnp.concatenate([aT[2], aT[3], aT[6], aT[7]]),
    ])

    loops = jnp.arange(n, dtype=jnp.int32)
    src = jnp.concatenate([edge_index[0].astype(jnp.int32), loops])
    dst = jnp.concatenate([edge_index[1].astype(jnp.int32), loops])
    e_tot = src.shape[0]
    fill = jnp.full((EP - e_tot,), N_PAD - 1, jnp.int32)
    src_p = jnp.concatenate([src, fill]).reshape(NT, NCHUNK, K)
    dst_p = jnp.concatenate([dst, fill]).reshape(NT, NCHUNK, K)

    num, den = _aggregate(hl, hr, a_planes, src_p, dst_p)

    rep = jnp.zeros((16, HC), jnp.float32)
    for c in range(2):
        for hh in range(2):
            head = 2 * c + hh
            rep = rep.at[8 * c + hh, 32 * head:32 * (head + 1)].set(1.0)
    out = _finalize(num, den, rep, bias.reshape(1, HC))
    return out[:n]


# speedup vs baseline: 68.8029x; 1.0461x over previous
import jax
import jax.numpy as jnp
from jax import lax
from jax.experimental import pallas as pl
from jax.experimental.pallas import tpu as pltpu
from jax.experimental.pallas import tpu_sc as plsc

N_PAD = 10016
D_IN = 128
HC = 128
HHC = HC // 2
H = 4
K = 48
NCHUNK = 432
NT = 16
ROWS_PT = N_PAD // NT
EP = NT * NCHUNK * K


def _proj_body(x_ref, w_ref, wa_ref, hl_ref, hr_ref, a_ref):
    xv = x_ref[...]
    xe = jnp.where(xv > 0, xv, jnp.exp(xv) - 1.0)
    w = w_ref[...]
    hl_ref[...] = jnp.dot(xe, w[:, :HHC], preferred_element_type=jnp.float32)
    hr_ref[...] = jnp.dot(xe, w[:, HHC:], preferred_element_type=jnp.float32)
    a_ref[...] = jax.lax.dot_general(
        wa_ref[...], xe, (((0,), (1,)), ((), ())),
        preferred_element_type=jnp.float32)


def _project(x_pad, W, Wa):
    blk = N_PAD
    return pl.pallas_call(
        _proj_body,
        grid=(N_PAD // blk,),
        in_specs=[
            pl.BlockSpec((blk, D_IN), lambda i: (i, 0)),
            pl.BlockSpec((D_IN, HC), lambda i: (0, 0)),
            pl.BlockSpec((D_IN, 8), lambda i: (0, 0)),
        ],
        out_specs=[
            pl.BlockSpec((blk, HHC), lambda i: (i, 0)),
            pl.BlockSpec((blk, HHC), lambda i: (i, 0)),
            pl.BlockSpec((8, blk), lambda i: (0, i)),
        ],
        out_shape=[
            jax.ShapeDtypeStruct((N_PAD, HHC), jnp.float32),
            jax.ShapeDtypeStruct((N_PAD, HHC), jnp.float32),
            jax.ShapeDtypeStruct((8, N_PAD), jnp.float32),
        ],
    )(x_pad, W, Wa)


def _agg_body(hl_hbm, hr_hbm, a_hbm, src_hbm, dst_hbm, num_hbm, den_hbm,
              a_vmem, src_vmem, dst_vmem, rows_buf, w_pl, w_rows, acc, den_acc, sem):
    cid = lax.axis_index("c")
    sid = lax.axis_index("s")
    zeros16 = jnp.zeros((16,), jnp.float32)
    iota16 = lax.iota(jnp.int32, 16)

    pltpu.sync_copy(a_hbm.at[cid], a_vmem)
    pltpu.sync_copy(src_hbm.at[sid], src_vmem)
    pltpu.sync_copy(dst_hbm.at[sid], dst_vmem)

    def _zero_row(r, _):
        for q in range(HHC // 16):
            rows_buf[r, pl.ds(q * 16, 16)] = zeros16
        return 0
    lax.fori_loop(0, K, _zero_row, 0)
    for g in range(K // 16):
        for cc in range(8):
            plsc.store_scatter(w_rows,
                               [g * 16 + iota16, jnp.full((16,), cc, jnp.int32)],
                               zeros16)
    for j in range(ROWS_PT // K):
        pltpu.sync_copy(rows_buf, acc.at[pl.ds(sid * ROWS_PT + j * K, K)])
        pltpu.sync_copy(w_rows, den_acc.at[pl.ds(sid * ROWS_PT + j * K, K)])
    _rem = ROWS_PT % K
    if _rem:
        pltpu.sync_copy(rows_buf.at[pl.ds(0, _rem)],
                        acc.at[pl.ds(sid * ROWS_PT + (ROWS_PT // K) * K, _rem)])
        pltpu.sync_copy(w_rows.at[pl.ds(0, _rem)],
                        den_acc.at[pl.ds(sid * ROWS_PT + (ROWS_PT // K) * K, _rem)])
    plsc.subcore_barrier()

    def _chunk(c, _):
        @pl.when(cid == 0)
        def _():
            pltpu.async_copy(hl_hbm.at[src_vmem.at[c]], rows_buf, sem)

        @pl.when(cid == 1)
        def _():
            pltpu.async_copy(hr_hbm.at[src_vmem.at[c]], rows_buf, sem)
        for g in range(K // 16):
            sv = src_vmem[c, pl.ds(g * 16, 16)]
            dv = dst_vmem[c, pl.ds(g * 16, 16)]
            for hh in range(2):
                asv = plsc.load_gather(a_vmem, [sv + hh * N_PAD])
                adv = plsc.load_gather(a_vmem, [dv + (2 + hh) * N_PAD])
                al = asv + adv
                al = jnp.maximum(al, al * 0.2)
                w = jnp.exp(al)
                w_pl[hh, pl.ds(g * 16, 16)] = w
                plsc.store_scatter(
                    w_rows,
                    [g * 16 + iota16, jnp.full((16,), hh, jnp.int32)],
                    w)
        pltpu.make_async_copy(hl_hbm.at[src_vmem.at[c]], rows_buf, sem).wait()
        for g in range(K // 16):
            w0 = w_pl[0, pl.ds(g * 16, 16)]
            w1 = w_pl[1, pl.ds(g * 16, 16)]
            for j in range(16):
                e = g * 16 + j
                for q in range(4):
                    ws = w0[j] if q < 2 else w1[j]
                    rows_buf[e, pl.ds(q * 16, 16)] = rows_buf[e, pl.ds(q * 16, 16)] * ws
        pltpu.sync_copy(rows_buf, acc.at[dst_vmem.at[c]], add=True)
        pltpu.sync_copy(w_rows, den_acc.at[dst_vmem.at[c]], add=True)
        return 0
    lax.fori_loop(0, NCHUNK, _chunk, 0)
    plsc.subcore_barrier()

    base = sid * ROWS_PT
    pltpu.sync_copy(acc.at[pl.ds(base, ROWS_PT)], num_hbm.at[cid, pl.ds(base, ROWS_PT)])
    pltpu.sync_copy(den_acc.at[pl.ds(base, ROWS_PT)], den_hbm.at[cid, pl.ds(base, ROWS_PT)])


def _aggregate(hl, hr, a_planes, src_p, dst_p):
    mesh = plsc.VectorSubcoreMesh(core_axis_name="c", subcore_axis_name="s",
                                  num_cores=2, num_subcores=NT)
    return pl.kernel(
        _agg_body,
        out_type=[
            jax.ShapeDtypeStruct((2, N_PAD, HHC), jnp.float32),
            jax.ShapeDtypeStruct((2, N_PAD, 8), jnp.float32),
        ],
        mesh=mesh,
        compiler_params=pltpu.CompilerParams(needs_layout_passes=False, use_tc_tiling_on_sc=False),
        scratch_types=[
            pltpu.VMEM((4 * N_PAD,), jnp.float32),
            pltpu.VMEM((NCHUNK, K), jnp.int32),
            pltpu.VMEM((NCHUNK, K), jnp.int32),
            pltpu.VMEM((K, HHC), jnp.float32),
            pltpu.VMEM((2, K), jnp.float32),
            pltpu.VMEM((K, 8), jnp.float32),
            pltpu.VMEM_SHARED((N_PAD, HHC), jnp.float32),
            pltpu.VMEM_SHARED((N_PAD, 8), jnp.float32),
            pltpu.SemaphoreType.DMA,
        ],
    )(hl, hr, a_planes, src_p, dst_p)


def _fin_body(n_ref, d_ref, rep_ref, b_ref, o_ref):
    num = jnp.concatenate([n_ref[0], n_ref[1]], axis=1)
    den = jnp.concatenate([d_ref[0], d_ref[1]], axis=1)
    denf = jnp.dot(den, rep_ref[...], preferred_element_type=jnp.float32)
    o_ref[...] = num / denf + b_ref[...]


def _finalize(num, den, Rep, bias2d):
    blk = 2504
    return pl.pallas_call(
        _fin_body,
        grid=(N_PAD // blk,),
        in_specs=[
            pl.BlockSpec((2, blk, HHC), lambda i: (0, i, 0)),
            pl.BlockSpec((2, blk, 8), lambda i: (0, i, 0)),
            pl.BlockSpec((16, HC), lambda i: (0, 0)),
            pl.BlockSpec((1, HC), lambda i: (0, 0)),
        ],
        out_specs=pl.BlockSpec((blk, HC), lambda i: (i, 0)),
        out_shape=jax.ShapeDtypeStruct((N_PAD, HC), jnp.float32),
    )(num, den, Rep, bias2d)


@jax.jit
def kernel(x, edge_index, W, att_src, att_dst, bias):
    n = x.shape[0]
    x_pad = jnp.zeros((N_PAD, D_IN), jnp.float32).at[:n].set(x)
    W3 = W.reshape(D_IN, H, HC // H)
    Ws = jnp.einsum("dhc,hc->dh", W3, att_src)
    Wd = jnp.einsum("dhc,hc->dh", W3, att_dst)
    Wa = jnp.stack([Ws[:, 0], Ws[:, 1], Wd[:, 0], Wd[:, 1],
                    Ws[:, 2], Ws[:, 3], Wd[:, 2], Wd[:, 3]], axis=1)

    hl, hr, a = _project(x_pad, W, Wa)
    a_planes = a.reshape(2, 4 * N_PAD)

    loops = jnp.arange(n, dtype=jnp.int32)
    src = jnp.concatenate([edge_index[0].astype(jnp.int32), loops])
    dst = jnp.concatenate([edge_index[1].astype(jnp.int32), loops])
    e_tot = src.shape[0]
    fill = jnp.full((EP - e_tot,), N_PAD - 1, jnp.int32)
    src_p = jnp.concatenate([src, fill]).reshape(NT, NCHUNK, K)
    dst_p = jnp.concatenate([dst, fill]).reshape(NT, NCHUNK, K)

    num, den = _aggregate(hl, hr, a_planes, src_p, dst_p)

    rep = jnp.zeros((16, HC), jnp.float32)
    for c in range(2):
        for hh in range(2):
            head = 2 * c + hh
            rep = rep.at[8 * c + hh, 32 * head:32 * (head + 1)].set(1.0)
    out = _finalize(num, den, rep, bias.reshape(1, HC))
    return out[:n]


# speedup vs baseline: 69.8756x; 1.0156x over previous
import jax
import jax.numpy as jnp
from jax import lax
from jax.experimental import pallas as pl
from jax.experimental.pallas import tpu as pltpu
from jax.experimental.pallas import tpu_sc as plsc

N_PAD = 10016
D_IN = 128
HC = 128
HHC = HC // 2
H = 4
K = 48
NCHUNK = 432
NT = 16
ROWS_PT = N_PAD // NT
EP = NT * NCHUNK * K


def _proj_body(x_ref, w_ref, wa_ref, hl_ref, hr_ref, a_ref):
    xv = x_ref[...]
    xe = jnp.where(xv > 0, xv, jnp.exp(xv) - 1.0)
    xe = jnp.concatenate(
        [xe, jnp.zeros((N_PAD - xe.shape[0], D_IN), jnp.float32)], axis=0)
    w = w_ref[...]
    hl_ref[...] = jnp.dot(xe, w[:, :HHC], preferred_element_type=jnp.float32)
    hr_ref[...] = jnp.dot(xe, w[:, HHC:], preferred_element_type=jnp.float32)
    a_ref[...] = jax.lax.dot_general(
        wa_ref[...], xe, (((0,), (1,)), ((), ())),
        preferred_element_type=jnp.float32)


def _project(x, W, Wa):
    n = x.shape[0]
    blk = N_PAD
    return pl.pallas_call(
        _proj_body,
        grid=(1,),
        in_specs=[
            pl.BlockSpec((n, D_IN), lambda i: (0, 0)),
            pl.BlockSpec((D_IN, HC), lambda i: (0, 0)),
            pl.BlockSpec((D_IN, 8), lambda i: (0, 0)),
        ],
        out_specs=[
            pl.BlockSpec((blk, HHC), lambda i: (i, 0)),
            pl.BlockSpec((blk, HHC), lambda i: (i, 0)),
            pl.BlockSpec((8, blk), lambda i: (0, i)),
        ],
        out_shape=[
            jax.ShapeDtypeStruct((N_PAD, HHC), jnp.float32),
            jax.ShapeDtypeStruct((N_PAD, HHC), jnp.float32),
            jax.ShapeDtypeStruct((8, N_PAD), jnp.float32),
        ],
    )(x, W, Wa)


def _agg_body(hl_hbm, hr_hbm, a_hbm, src_hbm, dst_hbm, num_hbm, den_hbm,
              a_vmem, src_vmem, dst_vmem, rows_buf, w_pl, w_rows, acc, den_acc, sem):
    cid = lax.axis_index("c")
    sid = lax.axis_index("s")
    zeros16 = jnp.zeros((16,), jnp.float32)
    iota16 = lax.iota(jnp.int32, 16)

    pltpu.sync_copy(a_hbm.at[cid], a_vmem)
    pltpu.sync_copy(src_hbm.at[sid], src_vmem)
    pltpu.sync_copy(dst_hbm.at[sid], dst_vmem)

    def _zero_row(r, _):
        for q in range(HHC // 16):
            rows_buf[r, pl.ds(q * 16, 16)] = zeros16
        return 0
    lax.fori_loop(0, K, _zero_row, 0)
    for g in range(K // 16):
        for cc in range(8):
            plsc.store_scatter(w_rows,
                               [g * 16 + iota16, jnp.full((16,), cc, jnp.int32)],
                               zeros16)
    for j in range(ROWS_PT // K):
        pltpu.sync_copy(rows_buf, acc.at[pl.ds(sid * ROWS_PT + j * K, K)])
        pltpu.sync_copy(w_rows, den_acc.at[pl.ds(sid * ROWS_PT + j * K, K)])
    _rem = ROWS_PT % K
    if _rem:
        pltpu.sync_copy(rows_buf.at[pl.ds(0, _rem)],
                        acc.at[pl.ds(sid * ROWS_PT + (ROWS_PT // K) * K, _rem)])
        pltpu.sync_copy(w_rows.at[pl.ds(0, _rem)],
                        den_acc.at[pl.ds(sid * ROWS_PT + (ROWS_PT // K) * K, _rem)])
    plsc.subcore_barrier()

    def _chunk(c, _):
        @pl.when(cid == 0)
        def _():
            pltpu.async_copy(hl_hbm.at[src_vmem.at[c]], rows_buf, sem)

        @pl.when(cid == 1)
        def _():
            pltpu.async_copy(hr_hbm.at[src_vmem.at[c]], rows_buf, sem)
        for g in range(K // 16):
            sv = src_vmem[c, pl.ds(g * 16, 16)]
            dv = dst_vmem[c, pl.ds(g * 16, 16)]
            for hh in range(2):
                asv = plsc.load_gather(a_vmem, [sv + hh * N_PAD])
                adv = plsc.load_gather(a_vmem, [dv + (2 + hh) * N_PAD])
                al = asv + adv
                al = jnp.maximum(al, al * 0.2)
                w = jnp.exp(al)
                w_pl[hh, pl.ds(g * 16, 16)] = w
                plsc.store_scatter(
                    w_rows,
                    [g * 16 + iota16, jnp.full((16,), hh, jnp.int32)],
                    w)
        pltpu.make_async_copy(hl_hbm.at[src_vmem.at[c]], rows_buf, sem).wait()
        for g in range(K // 16):
            w0 = w_pl[0, pl.ds(g * 16, 16)]
            w1 = w_pl[1, pl.ds(g * 16, 16)]
            for j in range(16):
                e = g * 16 + j
                for q in range(4):
                    ws = w0[j] if q < 2 else w1[j]
                    rows_buf[e, pl.ds(q * 16, 16)] = rows_buf[e, pl.ds(q * 16, 16)] * ws
        pltpu.sync_copy(rows_buf, acc.at[dst_vmem.at[c]], add=True)
        pltpu.sync_copy(w_rows, den_acc.at[dst_vmem.at[c]], add=True)
        return 0
    lax.fori_loop(0, NCHUNK, _chunk, 0)
    plsc.subcore_barrier()

    base = sid * ROWS_PT
    pltpu.sync_copy(acc.at[pl.ds(base, ROWS_PT)], num_hbm.at[cid, pl.ds(base, ROWS_PT)])
    pltpu.sync_copy(den_acc.at[pl.ds(base, ROWS_PT)], den_hbm.at[cid, pl.ds(base, ROWS_PT)])


def _aggregate(hl, hr, a_planes, src_p, dst_p):
    mesh = plsc.VectorSubcoreMesh(core_axis_name="c", subcore_axis_name="s",
                                  num_cores=2, num_subcores=NT)
    return pl.kernel(
        _agg_body,
        out_type=[
            jax.ShapeDtypeStruct((2, N_PAD, HHC), jnp.float32),
            jax.ShapeDtypeStruct((2, N_PAD, 8), jnp.float32),
        ],
        mesh=mesh,
        compiler_params=pltpu.CompilerParams(needs_layout_passes=False, use_tc_tiling_on_sc=False),
        scratch_types=[
            pltpu.VMEM((4 * N_PAD,), jnp.float32),
            pltpu.VMEM((NCHUNK, K), jnp.int32),
            pltpu.VMEM((NCHUNK, K), jnp.int32),
            pltpu.VMEM((K, HHC), jnp.float32),
            pltpu.VMEM((2, K), jnp.float32),
            pltpu.VMEM((K, 8), jnp.float32),
            pltpu.VMEM_SHARED((N_PAD, HHC), jnp.float32),
            pltpu.VMEM_SHARED((N_PAD, 8), jnp.float32),
            pltpu.SemaphoreType.DMA,
        ],
    )(hl, hr, a_planes, src_p, dst_p)


def _fin_body(n_ref, d_ref, rep_ref, b_ref, o_ref):
    nr = o_ref.shape[0]
    num = jnp.concatenate([n_ref[0, :nr], n_ref[1, :nr]], axis=1)
    den = jnp.concatenate([d_ref[0, :nr], d_ref[1, :nr]], axis=1)
    denf = jnp.dot(den, rep_ref[...], preferred_element_type=jnp.float32)
    o_ref[...] = num / denf + b_ref[...]


def _finalize(num, den, Rep, bias2d, n):
    return pl.pallas_call(
        _fin_body,
        grid=(1,),
        in_specs=[
            pl.BlockSpec((2, N_PAD, HHC), lambda i: (0, 0, 0)),
            pl.BlockSpec((2, N_PAD, 8), lambda i: (0, 0, 0)),
            pl.BlockSpec((16, HC), lambda i: (0, 0)),
            pl.BlockSpec((1, HC), lambda i: (0, 0)),
        ],
        out_specs=pl.BlockSpec((n, HC), lambda i: (0, 0)),
        out_shape=jax.ShapeDtypeStruct((n, HC), jnp.float32),
    )(num, den, Rep, bias2d)


@jax.jit
def kernel(x, edge_index, W, att_src, att_dst, bias):
    n = x.shape[0]
    W3 = W.reshape(D_IN, H, HC // H)
    Ws = jnp.einsum("dhc,hc->dh", W3, att_src)
    Wd = jnp.einsum("dhc,hc->dh", W3, att_dst)
    Wa = jnp.stack([Ws[:, 0], Ws[:, 1], Wd[:, 0], Wd[:, 1],
                    Ws[:, 2], Ws[:, 3], Wd[:, 2], Wd[:, 3]], axis=1)

    hl, hr, a = _project(x, W, Wa)
    a_planes = a.reshape(2, 4 * N_PAD)

    loops = jnp.arange(n, dtype=jnp.int32)
    src = jnp.concatenate([edge_index[0].astype(jnp.int32), loops])
    dst = jnp.concatenate([edge_index[1].astype(jnp.int32), loops])
    e_tot = src.shape[0]
    fill = jnp.full((EP - e_tot,), N_PAD - 1, jnp.int32)
    src_p = jnp.concatenate([src, fill]).reshape(NT, NCHUNK, K)
    dst_p = jnp.concatenate([dst, fill]).reshape(NT, NCHUNK, K)

    num, den = _aggregate(hl, hr, a_planes, src_p, dst_p)

    rep = jnp.zeros((16, HC), jnp.float32)
    for c in range(2):
        for hh in range(2):
            head = 2 * c + hh
            rep = rep.at[8 * c + hh, 32 * head:32 * (head + 1)].set(1.0)
    return _finalize(num, den, rep, bias.reshape(1, HC), n)
